# baseline (speedup 1.0000x reference)
import functools

import jax
import jax.numpy as jnp
from jax.experimental import pallas as pl
from jax.experimental.pallas import tpu as pltpu

N = 8192
NEV = 16
H = 64
S = 128
NBLK = N // S
INF_BITS = 0x7F800000


def _silu(x):
    return x * jax.nn.sigmoid(x)


def _vertex_kernel(xv_ref, w1_ref, b1_ref, w2_ref, b2_ref, w3_ref, b3_ref,
                   out_ref):
    x = xv_ref[...]
    h1 = _silu(jnp.dot(x, w1_ref[...], preferred_element_type=jnp.float32)
               + b1_ref[...])
    h2 = _silu(jnp.dot(h1, w2_ref[...], preferred_element_type=jnp.float32)
               + b2_ref[...])
    out_ref[...] = (jnp.dot(h2, w3_ref[...], preferred_element_type=jnp.float32)
                    + b3_ref[...])


def _pfc_enc_kernel(x_ref, nw1_ref, nb1_ref, nw2_ref, nb2_ref,
                    cw1_ref, cb1_ref, cw2_ref, cb2_ref,
                    a1_ref, e1b1_ref, b1w_ref,
                    enc_ref, u_ref, v_ref):
    x = x_ref[...]
    chg = jnp.dot(_silu(jnp.dot(x, cw1_ref[...],
                                preferred_element_type=jnp.float32)
                        + cb1_ref[...]),
                  cw2_ref[...], preferred_element_type=jnp.float32) + cb2_ref[...]
    xn = x[:, :11]
    neu = jnp.dot(_silu(jnp.dot(xn, nw1_ref[...],
                                preferred_element_type=jnp.float32)
                        + nb1_ref[...]),
                  nw2_ref[...], preferred_element_type=jnp.float32) + nb2_ref[...]
    cmask = (x[:, 10:11] != 0.0).astype(jnp.float32)
    enc = chg * cmask + neu * (1.0 - cmask)
    enc_ref[...] = enc
    u_ref[...] = (jnp.dot(enc, a1_ref[...], preferred_element_type=jnp.float32)
                  + e1b1_ref[...])
    v_ref[...] = jnp.dot(enc, b1w_ref[...], preferred_element_type=jnp.float32)


def _prep2_kernel(x_ref, f1_ref, a2_ref, e2b1_ref, b2w_ref,
                  cat_ref, u_ref, v_ref):
    x = x_ref[...]
    f1 = f1_ref[...]
    zeros = jnp.zeros((x.shape[0], 128 - 75), jnp.float32)
    cat = jnp.concatenate([x[:, :11], f1, zeros], axis=1)
    cat_ref[...] = cat
    u_ref[...] = (jnp.dot(cat, a2_ref[...], preferred_element_type=jnp.float32)
                  + e2b1_ref[...])
    v_ref[...] = jnp.dot(cat, b2w_ref[...], preferred_element_type=jnp.float32)


def _head_kernel(f_ref, w1_ref, b1_ref, w2_ref, b2_ref, w3_ref, b3_ref,
                 out_ref):
    f = f_ref[...]
    h1 = _silu(jnp.dot(f, w1_ref[...], preferred_element_type=jnp.float32)
               + b1_ref[...])
    h2 = _silu(jnp.dot(h1, w2_ref[...], preferred_element_type=jnp.float32)
               + b2_ref[...])
    o = jax.nn.sigmoid(jnp.dot(h2, w3_ref[...],
                               preferred_element_type=jnp.float32) + b3_ref[...])
    out_ref[...] = jnp.broadcast_to(o, (f.shape[0], 128))


def _conv_kernel(scal_ref, bt_ref, btr_ref, bs_ref, bsc_ref, xt_ref, xs_ref,
                 u_ref, v_ref, w2_ref, b2_ref, out_ref, d_ref, dt_ref,
                 *, k, T, C):
    i = pl.program_id(0)
    lo = scal_ref[2 * i]
    nblk = scal_ref[2 * i + 1]
    lo_blk = lo // S

    xt = xt_ref[...]
    bt = bt_ref[0]
    btr = btr_ref[0]
    u = u_ref[...]
    nt = jnp.sum(xt * xt, axis=1, keepdims=True)
    ntr = jnp.sum(xt * xt, axis=1)[None, :]

    def dist_body(b, carry):
        row0 = lo + b * S
        xsb = xs_ref[pl.ds(row0, S), :]
        bsb = bs_ref[lo_blk + b]
        bsc = bsc_ref[lo_blk + b]
        ns = jnp.sum(xsb * xsb, axis=1)[None, :]
        nsc = jnp.sum(xsb * xsb, axis=1, keepdims=True)
        d = nt - 2.0 * jax.lax.dot_general(
            xt, xsb, (((1,), (1,)), ((), ())),
            preferred_element_type=jnp.float32) + ns
        d = jnp.maximum(d, 0.0)
        d = jnp.where(bt == bsb, d, jnp.inf)
        d_ref[b] = d
        dt = nsc - 2.0 * jax.lax.dot_general(
            xsb, xt, (((1,), (1,)), ((), ())),
            preferred_element_type=jnp.float32) + ntr
        dt = jnp.maximum(dt, 0.0)
        dt = jnp.where(bsc == btr, dt, jnp.inf)
        dt_ref[b] = dt
        return carry

    jax.lax.fori_loop(0, nblk, dist_body, 0, unroll=False)

    def count3(t1, t2, t3):
        def cbody(b, c):
            c1, c2, c3 = c
            dtb = dt_ref[b]
            c1 = c1 + jnp.sum((dtb <= t1).astype(jnp.int32), axis=0,
                              keepdims=True)
            c2 = c2 + jnp.sum((dtb <= t2).astype(jnp.int32), axis=0,
                              keepdims=True)
            c3 = c3 + jnp.sum((dtb <= t3).astype(jnp.int32), axis=0,
                              keepdims=True)
            return c1, c2, c3
        z = jnp.zeros((1, T), jnp.int32)
        return jax.lax.fori_loop(0, nblk, cbody, (z, z, z), unroll=False)

    def sbody(_, state):
        blo, bhi = state
        q = jax.lax.div(bhi - blo, 4)
        m1 = blo + q
        m2 = m1 + q
        m3 = m2 + q
        c1, c2, c3 = count3(
            jax.lax.bitcast_convert_type(m1, jnp.float32),
            jax.lax.bitcast_convert_type(m2, jnp.float32),
            jax.lax.bitcast_convert_type(m3, jnp.float32))
        g1 = c1 >= k
        g2 = c2 >= k
        g3 = c3 >= k
        nhi = jnp.where(g1, m1, jnp.where(g2, m2, jnp.where(g3, m3, bhi)))
        nlo = jnp.where(g1, blo,
                        jnp.where(g2, m1 + 1, jnp.where(g3, m2 + 1, m3 + 1)))
        return nlo, nhi

    blo = jnp.zeros((1, T), jnp.int32)
    bhi = jnp.full((1, T), INF_BITS, jnp.int32)
    blo, bhi = jax.lax.fori_loop(0, 18, sbody, (blo, bhi), unroll=False)
    thresh_row = jax.lax.bitcast_convert_type(bhi, jnp.float32)
    thresh = jnp.transpose(thresh_row)

    def cnt_body(b, c):
        return c + jnp.sum((dt_ref[b] <= thresh_row).astype(jnp.float32),
                           axis=0, keepdims=True)
    cnt_row = jax.lax.fori_loop(0, nblk, cnt_body,
                                jnp.zeros((1, T), jnp.float32), unroll=False)
    cnt = jnp.transpose(cnt_row)

    ub = u.astype(jnp.bfloat16)

    def acc_body(b, acc):
        d = d_ref[b]
        mf = (d <= thresh).astype(jnp.bfloat16)
        row0 = lo + b * S
        for c in range(S // C):
            vc = v_ref[pl.ds(row0 + c * C, C), :].astype(jnp.bfloat16)
            mc = mf[:, c * C:(c + 1) * C]
            a = ub[:, None, :] + vc[None, :, :]
            s = a * jax.nn.sigmoid(a)
            acc = acc + (s * mc[:, :, None]).sum(axis=1, dtype=jnp.float32)
        return acc

    acc = jax.lax.fori_loop(
        0, nblk, acc_body, jnp.zeros((T, H), jnp.float32), unroll=False)
    feats = acc / cnt
    out_ref[...] = (jnp.dot(feats, w2_ref[...],
                            preferred_element_type=jnp.float32) + b2_ref[...])


def _edge_conv(scal, bt3, btr3, bs3, bsc3, xt, xs, u, v, w2, b2, *, k, T, C,
               interpret=False):
    n, d_feat = xt.shape
    ntiles = n // T
    grid_spec = pltpu.PrefetchScalarGridSpec(
        num_scalar_prefetch=1,
        grid=(ntiles,),
        in_specs=[
            pl.BlockSpec((1, T, 1), lambda i, s: (i, 0, 0)),
            pl.BlockSpec((1, 1, T), lambda i, s: (i, 0, 0)),
            pl.BlockSpec((NBLK, 1, S), lambda i, s: (0, 0, 0)),
            pl.BlockSpec((NBLK, S, 1), lambda i, s: (0, 0, 0)),
            pl.BlockSpec((T, d_feat), lambda i, s: (i, 0)),
            pl.BlockSpec((n, d_feat), lambda i, s: (0, 0)),
            pl.BlockSpec((T, H), lambda i, s: (i, 0)),
            pl.BlockSpec((n, H), lambda i, s: (0, 0)),
            pl.BlockSpec((H, H), lambda i, s: (0, 0)),
            pl.BlockSpec((1, H), lambda i, s: (0, 0)),
        ],
        out_specs=pl.BlockSpec((T, H), lambda i, s: (i, 0)),
        scratch_shapes=[pltpu.VMEM((NBLK, T, S), jnp.float32),
                        pltpu.VMEM((NBLK, S, T), jnp.float32)],
    )
    return pl.pallas_call(
        functools.partial(_conv_kernel, k=k, T=T, C=C),
        grid_spec=grid_spec,
        out_shape=jax.ShapeDtypeStruct((n, H), jnp.float32),
        compiler_params=pltpu.CompilerParams(
            dimension_semantics=("arbitrary",)),
        interpret=interpret,
    )(scal, bt3, btr3, bs3, bsc3, xt, xs, u, v, w2, b2)


def _tile_specs(t, shapes):
    specs = [pl.BlockSpec((t, shapes[0][1]), lambda i: (i, 0))]
    for s in shapes[1:]:
        specs.append(pl.BlockSpec(s, lambda i, _s=s: tuple(0 for _ in _s)))
    return specs


def kernel(x_pfc, x_vtx, batch_pfc, batch_vtx,
           vW1, vb1, vW2, vb2, vW3, vb3,
           nW1, nb1, nW2, nb2,
           cW1, cb1, cW2, cb2,
           e1W1, e1b1, e1W2, e1b2,
           e2W1, e2b1, e2W2, e2b2,
           oW1, ob1, oW2, ob2, oW3, ob3):
    k1, k2 = 32, 16
    T = 64
    C = 128

    f32 = jnp.float32
    batch_pfc_i = batch_pfc.astype(jnp.int32)

    x_vtx_enc = pl.pallas_call(
        _vertex_kernel,
        out_shape=jax.ShapeDtypeStruct((x_vtx.shape[0], H), f32),
    )(x_vtx, vW1, vb1[None, :], vW2, vb2[None, :], vW3, vb3[None, :])

    A1 = e1W1[:H] - e1W1[H:]
    B1 = e1W1[H:]
    TP = 512
    enc, u1, v1 = pl.pallas_call(
        _pfc_enc_kernel,
        grid=(N // TP,),
        in_specs=_tile_specs(TP, [(TP, 12), nW1.shape, (1, 32), nW2.shape,
                                  (1, H), cW1.shape, (1, 32), cW2.shape,
                                  (1, H), A1.shape, (1, H), B1.shape]),
        out_specs=[pl.BlockSpec((TP, H), lambda i: (i, 0))] * 3,
        out_shape=[jax.ShapeDtypeStruct((N, H), f32)] * 3,
    )(x_pfc, nW1, nb1[None, :], nW2, nb2[None, :],
      cW1, cb1[None, :], cW2, cb2[None, :],
      A1, e1b1[None, :], B1)

    starts = jnp.searchsorted(batch_pfc_i,
                              jnp.arange(NEV + 1, dtype=jnp.int32)
                              ).astype(jnp.int32)
    ntiles = N // T
    tidx = jnp.arange(ntiles, dtype=jnp.int32)
    first_b = batch_pfc_i[tidx * T]
    last_b = batch_pfc_i[(tidx + 1) * T - 1]
    span_lo = starts[first_b]
    span_hi = starts[last_b + 1]
    lo_row = (span_lo // S) * S
    nblk = (span_hi - lo_row + S - 1) // S
    nblk = jnp.maximum(nblk, 1)
    scal = jnp.stack([lo_row, nblk], axis=1).reshape(-1)

    bt3 = batch_pfc_i.reshape(ntiles, T, 1)
    btr3 = batch_pfc_i.reshape(ntiles, 1, T)
    bs3 = batch_pfc_i.reshape(NBLK, 1, S)
    bsc3 = batch_pfc_i.reshape(NBLK, S, 1)

    feats1 = _edge_conv(scal, bt3, btr3, bs3, bsc3, enc, enc, u1, v1,
                        e1W2, e1b2[None, :], k=k1, T=T, C=C)

    A2f = e2W1[:75] - e2W1[75:]
    B2f = e2W1[75:]
    pad = jnp.zeros((128 - 75, H), f32)
    A2 = jnp.concatenate([A2f, pad], axis=0)
    B2 = jnp.concatenate([B2f, pad], axis=0)
    cat, u2, v2 = pl.pallas_call(
        _prep2_kernel,
        grid=(N // TP,),
        in_specs=[pl.BlockSpec((TP, 12), lambda i: (i, 0)),
                  pl.BlockSpec((TP, H), lambda i: (i, 0)),
                  pl.BlockSpec(A2.shape, lambda i: (0, 0)),
                  pl.BlockSpec((1, H), lambda i: (0, 0)),
                  pl.BlockSpec(B2.shape, lambda i: (0, 0))],
        out_specs=[pl.BlockSpec((TP, 128), lambda i: (i, 0)),
                   pl.BlockSpec((TP, H), lambda i: (i, 0)),
                   pl.BlockSpec((TP, H), lambda i: (i, 0))],
        out_shape=[jax.ShapeDtypeStruct((N, 128), f32),
                   jax.ShapeDtypeStruct((N, H), f32),
                   jax.ShapeDtypeStruct((N, H), f32)],
    )(x_pfc, feats1, A2, e2b1[None, :], B2)

    charged_batch = jnp.where(x_pfc[:, 10] != 0.0, batch_pfc_i,
                              jnp.int32(-1))
    bs3c = charged_batch.reshape(NBLK, 1, S)
    bsc3c = charged_batch.reshape(NBLK, S, 1)
    feats2 = _edge_conv(scal, bt3, btr3, bs3c, bsc3c, cat, cat, u2, v2,
                        e2W2, e2b2[None, :], k=k2, T=T, C=C)

    out_pad = pl.pallas_call(
        _head_kernel,
        grid=(N // TP,),
        in_specs=_tile_specs(TP, [(TP, H), oW1.shape, (1, 16), oW2.shape,
                                  (1, 4), oW3.shape, (1, 1)]),
        out_specs=pl.BlockSpec((TP, 128), lambda i: (i, 0)),
        out_shape=jax.ShapeDtypeStruct((N, 128), f32),
    )(feats2, oW1, ob1[None, :], oW2, ob2[None, :], oW3, ob3[None, :])
    out = out_pad[:, :1]

    return (out, batch_pfc, feats1, x_vtx_enc)


# speedup vs baseline: 1.1791x; 1.1791x over previous
import functools

import jax
import jax.numpy as jnp
from jax.experimental import pallas as pl
from jax.experimental.pallas import tpu as pltpu

N = 8192
NEV = 16
H = 64
S = 256
NBLK = N // S
INF_BITS = 0x7F800000


def _silu(x):
    return x * jax.nn.sigmoid(x)


def _vertex_kernel(xv_ref, w1_ref, b1_ref, w2_ref, b2_ref, w3_ref, b3_ref,
                   out_ref):
    x = xv_ref[...]
    h1 = _silu(jnp.dot(x, w1_ref[...], preferred_element_type=jnp.float32)
               + b1_ref[...])
    h2 = _silu(jnp.dot(h1, w2_ref[...], preferred_element_type=jnp.float32)
               + b2_ref[...])
    out_ref[...] = (jnp.dot(h2, w3_ref[...], preferred_element_type=jnp.float32)
                    + b3_ref[...])


def _pfc_enc_kernel(x_ref, nw1_ref, nb1_ref, nw2_ref, nb2_ref,
                    cw1_ref, cb1_ref, cw2_ref, cb2_ref,
                    a1_ref, e1b1_ref, b1w_ref,
                    enc_ref, u_ref, v_ref):
    x = x_ref[...]
    chg = jnp.dot(_silu(jnp.dot(x, cw1_ref[...],
                                preferred_element_type=jnp.float32)
                        + cb1_ref[...]),
                  cw2_ref[...], preferred_element_type=jnp.float32) + cb2_ref[...]
    xn = x[:, :11]
    neu = jnp.dot(_silu(jnp.dot(xn, nw1_ref[...],
                                preferred_element_type=jnp.float32)
                        + nb1_ref[...]),
                  nw2_ref[...], preferred_element_type=jnp.float32) + nb2_ref[...]
    cmask = (x[:, 10:11] != 0.0).astype(jnp.float32)
    enc = chg * cmask + neu * (1.0 - cmask)
    enc_ref[...] = enc
    u_ref[...] = (jnp.dot(enc, a1_ref[...], preferred_element_type=jnp.float32)
                  + e1b1_ref[...])
    v_ref[...] = jnp.dot(enc, b1w_ref[...], preferred_element_type=jnp.float32)


def _prep2_kernel(x_ref, f1_ref, a2_ref, e2b1_ref, b2w_ref,
                  cat_ref, u_ref, v_ref):
    x = x_ref[...]
    f1 = f1_ref[...]
    zeros = jnp.zeros((x.shape[0], 128 - 75), jnp.float32)
    cat = jnp.concatenate([x[:, :11], f1, zeros], axis=1)
    cat_ref[...] = cat
    u_ref[...] = (jnp.dot(cat, a2_ref[...], preferred_element_type=jnp.float32)
                  + e2b1_ref[...])
    v_ref[...] = jnp.dot(cat, b2w_ref[...], preferred_element_type=jnp.float32)


def _head_kernel(f_ref, w1_ref, b1_ref, w2_ref, b2_ref, w3_ref, b3_ref,
                 out_ref):
    f = f_ref[...]
    h1 = _silu(jnp.dot(f, w1_ref[...], preferred_element_type=jnp.float32)
               + b1_ref[...])
    h2 = _silu(jnp.dot(h1, w2_ref[...], preferred_element_type=jnp.float32)
               + b2_ref[...])
    o = jax.nn.sigmoid(jnp.dot(h2, w3_ref[...],
                               preferred_element_type=jnp.float32) + b3_ref[...])
    out_ref[...] = jnp.broadcast_to(o, (f.shape[0], 128))


def _conv_kernel(scal_ref, bt_ref, btr_ref, bs_ref, bsc_ref, xt_ref, xs_ref,
                 u_ref, v_ref, w2_ref, b2_ref, out_ref, d_ref, dt_ref,
                 *, k, T, C):
    i = pl.program_id(0)
    lo = scal_ref[2 * i]
    nblk = scal_ref[2 * i + 1]
    lo_blk = lo // S

    xt = xt_ref[...]
    bt = bt_ref[0]
    btr = btr_ref[0]
    u = u_ref[...]
    nt = jnp.sum(xt * xt, axis=1, keepdims=True)
    ntr = jnp.sum(xt * xt, axis=1)[None, :]

    def dist_body(b, carry):
        row0 = lo + b * S
        xsb = xs_ref[pl.ds(row0, S), :]
        bsb = bs_ref[lo_blk + b]
        bsc = bsc_ref[lo_blk + b]
        ns = jnp.sum(xsb * xsb, axis=1)[None, :]
        nsc = jnp.sum(xsb * xsb, axis=1, keepdims=True)
        d = nt - 2.0 * jax.lax.dot_general(
            xt, xsb, (((1,), (1,)), ((), ())),
            preferred_element_type=jnp.float32) + ns
        d = jnp.maximum(d, 0.0)
        d = jnp.where(bt == bsb, d, jnp.inf)
        d_ref[b] = d
        dt = nsc - 2.0 * jax.lax.dot_general(
            xsb, xt, (((1,), (1,)), ((), ())),
            preferred_element_type=jnp.float32) + ntr
        dt = jnp.maximum(dt, 0.0)
        dt = jnp.where(bsc == btr, dt, jnp.inf)
        dt_ref[b] = dt
        return carry

    jax.lax.fori_loop(0, nblk, dist_body, 0, unroll=False)

    def count3(t1, t2, t3):
        def cbody(b, c):
            c1, c2, c3 = c
            dtb = dt_ref[b]
            c1 = c1 + jnp.sum((dtb <= t1).astype(jnp.int32), axis=0,
                              keepdims=True)
            c2 = c2 + jnp.sum((dtb <= t2).astype(jnp.int32), axis=0,
                              keepdims=True)
            c3 = c3 + jnp.sum((dtb <= t3).astype(jnp.int32), axis=0,
                              keepdims=True)
            return c1, c2, c3
        z = jnp.zeros((1, T), jnp.int32)
        return jax.lax.fori_loop(0, nblk, cbody, (z, z, z), unroll=False)

    def sbody(_, state):
        blo, bhi = state
        q = jax.lax.div(bhi - blo, 4)
        m1 = blo + q
        m2 = m1 + q
        m3 = m2 + q
        c1, c2, c3 = count3(
            jax.lax.bitcast_convert_type(m1, jnp.float32),
            jax.lax.bitcast_convert_type(m2, jnp.float32),
            jax.lax.bitcast_convert_type(m3, jnp.float32))
        g1 = c1 >= k
        g2 = c2 >= k
        g3 = c3 >= k
        nhi = jnp.where(g1, m1, jnp.where(g2, m2, jnp.where(g3, m3, bhi)))
        nlo = jnp.where(g1, blo,
                        jnp.where(g2, m1 + 1, jnp.where(g3, m2 + 1, m3 + 1)))
        return nlo, nhi

    blo = jnp.zeros((1, T), jnp.int32)
    bhi = jnp.full((1, T), INF_BITS, jnp.int32)
    blo, bhi = jax.lax.fori_loop(0, 18, sbody, (blo, bhi), unroll=False)
    thresh_row = jax.lax.bitcast_convert_type(bhi, jnp.float32)
    thresh = jnp.transpose(thresh_row)

    def cnt_body(b, c):
        return c + jnp.sum((dt_ref[b] <= thresh_row).astype(jnp.float32),
                           axis=0, keepdims=True)
    cnt_row = jax.lax.fori_loop(0, nblk, cnt_body,
                                jnp.zeros((1, T), jnp.float32), unroll=False)
    cnt = jnp.transpose(cnt_row)

    ub = u.astype(jnp.bfloat16)

    def acc_body(b, acc):
        d = d_ref[b]
        mf = (d <= thresh).astype(jnp.bfloat16)
        row0 = lo + b * S
        for c in range(S // C):
            vc = v_ref[pl.ds(row0 + c * C, C), :].astype(jnp.bfloat16)
            mc = mf[:, c * C:(c + 1) * C]
            a = ub[:, None, :] + vc[None, :, :]
            s = a * jax.nn.sigmoid(a)
            acc = acc + (s * mc[:, :, None]).sum(axis=1, dtype=jnp.float32)
        return acc

    acc = jax.lax.fori_loop(
        0, nblk, acc_body, jnp.zeros((T, H), jnp.float32), unroll=False)
    feats = acc / cnt
    out_ref[...] = (jnp.dot(feats, w2_ref[...],
                            preferred_element_type=jnp.float32) + b2_ref[...])


def _edge_conv(scal, bt3, btr3, bs3, bsc3, xt, xs, u, v, w2, b2, *, k, T, C,
               interpret=False):
    n, d_feat = xt.shape
    ntiles = n // T
    grid_spec = pltpu.PrefetchScalarGridSpec(
        num_scalar_prefetch=1,
        grid=(ntiles,),
        in_specs=[
            pl.BlockSpec((1, T, 1), lambda i, s: (i, 0, 0)),
            pl.BlockSpec((1, 1, T), lambda i, s: (i, 0, 0)),
            pl.BlockSpec((NBLK, 1, S), lambda i, s: (0, 0, 0)),
            pl.BlockSpec((NBLK, S, 1), lambda i, s: (0, 0, 0)),
            pl.BlockSpec((T, d_feat), lambda i, s: (i, 0)),
            pl.BlockSpec((n, d_feat), lambda i, s: (0, 0)),
            pl.BlockSpec((T, H), lambda i, s: (i, 0)),
            pl.BlockSpec((n, H), lambda i, s: (0, 0)),
            pl.BlockSpec((H, H), lambda i, s: (0, 0)),
            pl.BlockSpec((1, H), lambda i, s: (0, 0)),
        ],
        out_specs=pl.BlockSpec((T, H), lambda i, s: (i, 0)),
        scratch_shapes=[pltpu.VMEM((NBLK, T, S), jnp.float32),
                        pltpu.VMEM((NBLK, S, T), jnp.float32)],
    )
    return pl.pallas_call(
        functools.partial(_conv_kernel, k=k, T=T, C=C),
        grid_spec=grid_spec,
        out_shape=jax.ShapeDtypeStruct((n, H), jnp.float32),
        compiler_params=pltpu.CompilerParams(
            dimension_semantics=("arbitrary",)),
        interpret=interpret,
    )(scal, bt3, btr3, bs3, bsc3, xt, xs, u, v, w2, b2)


def _tile_specs(t, shapes):
    specs = [pl.BlockSpec((t, shapes[0][1]), lambda i: (i, 0))]
    for s in shapes[1:]:
        specs.append(pl.BlockSpec(s, lambda i, _s=s: tuple(0 for _ in _s)))
    return specs


def kernel(x_pfc, x_vtx, batch_pfc, batch_vtx,
           vW1, vb1, vW2, vb2, vW3, vb3,
           nW1, nb1, nW2, nb2,
           cW1, cb1, cW2, cb2,
           e1W1, e1b1, e1W2, e1b2,
           e2W1, e2b1, e2W2, e2b2,
           oW1, ob1, oW2, ob2, oW3, ob3):
    k1, k2 = 32, 16
    T = 256
    C = 128

    f32 = jnp.float32
    batch_pfc_i = batch_pfc.astype(jnp.int32)

    x_vtx_enc = pl.pallas_call(
        _vertex_kernel,
        out_shape=jax.ShapeDtypeStruct((x_vtx.shape[0], H), f32),
    )(x_vtx, vW1, vb1[None, :], vW2, vb2[None, :], vW3, vb3[None, :])

    A1 = e1W1[:H] - e1W1[H:]
    B1 = e1W1[H:]
    TP = 512
    enc, u1, v1 = pl.pallas_call(
        _pfc_enc_kernel,
        grid=(N // TP,),
        in_specs=_tile_specs(TP, [(TP, 12), nW1.shape, (1, 32), nW2.shape,
                                  (1, H), cW1.shape, (1, 32), cW2.shape,
                                  (1, H), A1.shape, (1, H), B1.shape]),
        out_specs=[pl.BlockSpec((TP, H), lambda i: (i, 0))] * 3,
        out_shape=[jax.ShapeDtypeStruct((N, H), f32)] * 3,
    )(x_pfc, nW1, nb1[None, :], nW2, nb2[None, :],
      cW1, cb1[None, :], cW2, cb2[None, :],
      A1, e1b1[None, :], B1)

    starts = jnp.searchsorted(batch_pfc_i,
                              jnp.arange(NEV + 1, dtype=jnp.int32)
                              ).astype(jnp.int32)
    ntiles = N // T
    tidx = jnp.arange(ntiles, dtype=jnp.int32)
    first_b = batch_pfc_i[tidx * T]
    last_b = batch_pfc_i[(tidx + 1) * T - 1]
    span_lo = starts[first_b]
    span_hi = starts[last_b + 1]
    lo_row = (span_lo // S) * S
    nblk = (span_hi - lo_row + S - 1) // S
    nblk = jnp.maximum(nblk, 1)
    scal = jnp.stack([lo_row, nblk], axis=1).reshape(-1)

    bt3 = batch_pfc_i.reshape(ntiles, T, 1)
    btr3 = batch_pfc_i.reshape(ntiles, 1, T)
    bs3 = batch_pfc_i.reshape(NBLK, 1, S)
    bsc3 = batch_pfc_i.reshape(NBLK, S, 1)

    feats1 = _edge_conv(scal, bt3, btr3, bs3, bsc3, enc, enc, u1, v1,
                        e1W2, e1b2[None, :], k=k1, T=T, C=C)

    A2f = e2W1[:75] - e2W1[75:]
    B2f = e2W1[75:]
    pad = jnp.zeros((128 - 75, H), f32)
    A2 = jnp.concatenate([A2f, pad], axis=0)
    B2 = jnp.concatenate([B2f, pad], axis=0)
    cat, u2, v2 = pl.pallas_call(
        _prep2_kernel,
        grid=(N // TP,),
        in_specs=[pl.BlockSpec((TP, 12), lambda i: (i, 0)),
                  pl.BlockSpec((TP, H), lambda i: (i, 0)),
                  pl.BlockSpec(A2.shape, lambda i: (0, 0)),
                  pl.BlockSpec((1, H), lambda i: (0, 0)),
                  pl.BlockSpec(B2.shape, lambda i: (0, 0))],
        out_specs=[pl.BlockSpec((TP, 128), lambda i: (i, 0)),
                   pl.BlockSpec((TP, H), lambda i: (i, 0)),
                   pl.BlockSpec((TP, H), lambda i: (i, 0))],
        out_shape=[jax.ShapeDtypeStruct((N, 128), f32),
                   jax.ShapeDtypeStruct((N, H), f32),
                   jax.ShapeDtypeStruct((N, H), f32)],
    )(x_pfc, feats1, A2, e2b1[None, :], B2)

    charged_batch = jnp.where(x_pfc[:, 10] != 0.0, batch_pfc_i,
                              jnp.int32(-1))
    bs3c = charged_batch.reshape(NBLK, 1, S)
    bsc3c = charged_batch.reshape(NBLK, S, 1)
    feats2 = _edge_conv(scal, bt3, btr3, bs3c, bsc3c, cat, cat, u2, v2,
                        e2W2, e2b2[None, :], k=k2, T=T, C=C)

    out_pad = pl.pallas_call(
        _head_kernel,
        grid=(N // TP,),
        in_specs=_tile_specs(TP, [(TP, H), oW1.shape, (1, 16), oW2.shape,
                                  (1, 4), oW3.shape, (1, 1)]),
        out_specs=pl.BlockSpec((TP, 128), lambda i: (i, 0)),
        out_shape=jax.ShapeDtypeStruct((N, 128), f32),
    )(feats2, oW1, ob1[None, :], oW2, ob2[None, :], oW3, ob3[None, :])
    out = out_pad[:, :1]

    return (out, batch_pfc, feats1, x_vtx_enc)


# speedup vs baseline: 1.2340x; 1.0466x over previous
import functools

import jax
import jax.numpy as jnp
from jax.experimental import pallas as pl
from jax.experimental.pallas import tpu as pltpu

N = 8192
NEV = 16
H = 64
S = 256
NBLK = N // S
INF_BITS = 0x7F800000


def _silu(x):
    return x * jax.nn.sigmoid(x)


def _vertex_kernel(xv_ref, w1_ref, b1_ref, w2_ref, b2_ref, w3_ref, b3_ref,
                   out_ref):
    x = xv_ref[...]
    h1 = _silu(jnp.dot(x, w1_ref[...], preferred_element_type=jnp.float32)
               + b1_ref[...])
    h2 = _silu(jnp.dot(h1, w2_ref[...], preferred_element_type=jnp.float32)
               + b2_ref[...])
    out_ref[...] = (jnp.dot(h2, w3_ref[...], preferred_element_type=jnp.float32)
                    + b3_ref[...])


def _pfc_enc_kernel(x_ref, nw1_ref, nb1_ref, nw2_ref, nb2_ref,
                    cw1_ref, cb1_ref, cw2_ref, cb2_ref,
                    a1_ref, e1b1_ref, b1w_ref,
                    enc_ref, u_ref, v_ref):
    x = x_ref[...]
    chg = jnp.dot(_silu(jnp.dot(x, cw1_ref[...],
                                preferred_element_type=jnp.float32)
                        + cb1_ref[...]),
                  cw2_ref[...], preferred_element_type=jnp.float32) + cb2_ref[...]
    xn = x[:, :11]
    neu = jnp.dot(_silu(jnp.dot(xn, nw1_ref[...],
                                preferred_element_type=jnp.float32)
                        + nb1_ref[...]),
                  nw2_ref[...], preferred_element_type=jnp.float32) + nb2_ref[...]
    cmask = (x[:, 10:11] != 0.0).astype(jnp.float32)
    enc = chg * cmask + neu * (1.0 - cmask)
    enc_ref[...] = enc
    u_ref[...] = (jnp.dot(enc, a1_ref[...], preferred_element_type=jnp.float32)
                  + e1b1_ref[...])
    v_ref[...] = jnp.dot(enc, b1w_ref[...], preferred_element_type=jnp.float32)


def _prep2_kernel(x_ref, f1_ref, a2_ref, e2b1_ref, b2w_ref,
                  cat_ref, u_ref, v_ref):
    x = x_ref[...]
    f1 = f1_ref[...]
    zeros = jnp.zeros((x.shape[0], 128 - 75), jnp.float32)
    cat = jnp.concatenate([x[:, :11], f1, zeros], axis=1)
    cat_ref[...] = cat
    u_ref[...] = (jnp.dot(cat, a2_ref[...], preferred_element_type=jnp.float32)
                  + e2b1_ref[...])
    v_ref[...] = jnp.dot(cat, b2w_ref[...], preferred_element_type=jnp.float32)


def _head_kernel(f_ref, w1_ref, b1_ref, w2_ref, b2_ref, w3_ref, b3_ref,
                 out_ref):
    f = f_ref[...]
    h1 = _silu(jnp.dot(f, w1_ref[...], preferred_element_type=jnp.float32)
               + b1_ref[...])
    h2 = _silu(jnp.dot(h1, w2_ref[...], preferred_element_type=jnp.float32)
               + b2_ref[...])
    o = jax.nn.sigmoid(jnp.dot(h2, w3_ref[...],
                               preferred_element_type=jnp.float32) + b3_ref[...])
    out_ref[...] = jnp.broadcast_to(o, (f.shape[0], 128))


def _conv_kernel(scal_ref, bt_ref, btr_ref, bs_ref, bsc_ref, xt_ref, xs_ref,
                 u_ref, v_ref, w2_ref, b2_ref, out_ref, d_ref, dt_ref,
                 *, k, T, C):
    i = pl.program_id(0)
    lo = scal_ref[2 * i]
    nblk = scal_ref[2 * i + 1]
    lo_blk = lo // S

    xt = xt_ref[...]
    bt = bt_ref[0]
    btr = btr_ref[0]
    u = u_ref[...]
    nt = jnp.sum(xt * xt, axis=1, keepdims=True)
    ntr = jnp.sum(xt * xt, axis=1)[None, :]

    def dist_body(b, carry):
        row0 = lo + b * S
        xsb = xs_ref[pl.ds(row0, S), :]
        bsb = bs_ref[lo_blk + b]
        bsc = bsc_ref[lo_blk + b]
        ns = jnp.sum(xsb * xsb, axis=1)[None, :]
        nsc = jnp.sum(xsb * xsb, axis=1, keepdims=True)
        d = nt - 2.0 * jax.lax.dot_general(
            xt, xsb, (((1,), (1,)), ((), ())),
            preferred_element_type=jnp.float32) + ns
        d = jnp.maximum(d, 0.0)
        d = jnp.where(bt == bsb, d, jnp.inf)
        d_ref[b] = d
        dt = nsc - 2.0 * jax.lax.dot_general(
            xsb, xt, (((1,), (1,)), ((), ())),
            preferred_element_type=jnp.float32) + ntr
        dt = jnp.maximum(dt, 0.0)
        dt = jnp.where(bsc == btr, dt, jnp.inf)
        dt_ref[b] = dt
        return carry

    jax.lax.fori_loop(0, nblk, dist_body, 0, unroll=False)

    def count3(t1, t2, t3):
        def cbody(b, c):
            c1, c2, c3 = c
            dtb = dt_ref[b]
            c1 = c1 + jnp.sum((dtb <= t1).astype(jnp.int32), axis=0,
                              keepdims=True)
            c2 = c2 + jnp.sum((dtb <= t2).astype(jnp.int32), axis=0,
                              keepdims=True)
            c3 = c3 + jnp.sum((dtb <= t3).astype(jnp.int32), axis=0,
                              keepdims=True)
            return c1, c2, c3
        z = jnp.zeros((1, T), jnp.int32)
        return jax.lax.fori_loop(0, nblk, cbody, (z, z, z), unroll=False)

    def sbody(_, state):
        blo, bhi = state
        q = jax.lax.div(bhi - blo, 4)
        m1 = blo + q
        m2 = m1 + q
        m3 = m2 + q
        c1, c2, c3 = count3(
            jax.lax.bitcast_convert_type(m1, jnp.float32),
            jax.lax.bitcast_convert_type(m2, jnp.float32),
            jax.lax.bitcast_convert_type(m3, jnp.float32))
        g1 = c1 >= k
        g2 = c2 >= k
        g3 = c3 >= k
        nhi = jnp.where(g1, m1, jnp.where(g2, m2, jnp.where(g3, m3, bhi)))
        nlo = jnp.where(g1, blo,
                        jnp.where(g2, m1 + 1, jnp.where(g3, m2 + 1, m3 + 1)))
        return nlo, nhi

    blo = jnp.zeros((1, T), jnp.int32)
    bhi = jnp.full((1, T), INF_BITS, jnp.int32)
    blo, bhi = jax.lax.fori_loop(0, 18, sbody, (blo, bhi), unroll=False)
    thresh_row = jax.lax.bitcast_convert_type(bhi, jnp.float32)
    thresh = jnp.transpose(thresh_row)

    def cnt_body(b, c):
        return c + jnp.sum((dt_ref[b] <= thresh_row).astype(jnp.float32),
                           axis=0, keepdims=True)
    cnt_row = jax.lax.fori_loop(0, nblk, cnt_body,
                                jnp.zeros((1, T), jnp.float32), unroll=False)
    cnt = jnp.transpose(cnt_row)

    ub = u.astype(jnp.bfloat16)

    def acc_body(b, acc):
        d = d_ref[b]
        mf = (d <= thresh).astype(jnp.bfloat16)
        row0 = lo + b * S
        for c in range(S // C):
            vc = v_ref[pl.ds(row0 + c * C, C), :].astype(jnp.bfloat16)
            mc = mf[:, c * C:(c + 1) * C]
            a = ub[:, None, :] + vc[None, :, :]
            s = a * jax.nn.sigmoid(a)
            acc = acc + (s * mc[:, :, None]).sum(axis=1, dtype=jnp.float32)
        return acc

    acc = jax.lax.fori_loop(
        0, nblk, acc_body, jnp.zeros((T, H), jnp.float32), unroll=False)
    feats = acc / cnt
    out_ref[...] = (jnp.dot(feats, w2_ref[...],
                            preferred_element_type=jnp.float32) + b2_ref[...])


def _edge_conv(scal, bt3, btr3, bs3, bsc3, xt, xs, u, v, w2, b2, *, k, T, C,
               interpret=False):
    n, d_feat = xt.shape
    ntiles = n // T
    grid_spec = pltpu.PrefetchScalarGridSpec(
        num_scalar_prefetch=1,
        grid=(ntiles,),
        in_specs=[
            pl.BlockSpec((1, T, 1), lambda i, s: (i, 0, 0)),
            pl.BlockSpec((1, 1, T), lambda i, s: (i, 0, 0)),
            pl.BlockSpec((NBLK, 1, S), lambda i, s: (0, 0, 0)),
            pl.BlockSpec((NBLK, S, 1), lambda i, s: (0, 0, 0)),
            pl.BlockSpec((T, d_feat), lambda i, s: (i, 0)),
            pl.BlockSpec((n, d_feat), lambda i, s: (0, 0)),
            pl.BlockSpec((T, H), lambda i, s: (i, 0)),
            pl.BlockSpec((n, H), lambda i, s: (0, 0)),
            pl.BlockSpec((H, H), lambda i, s: (0, 0)),
            pl.BlockSpec((1, H), lambda i, s: (0, 0)),
        ],
        out_specs=pl.BlockSpec((T, H), lambda i, s: (i, 0)),
        scratch_shapes=[pltpu.VMEM((NBLK, T, S), jnp.float32),
                        pltpu.VMEM((NBLK, S, T), jnp.float32)],
    )
    return pl.pallas_call(
        functools.partial(_conv_kernel, k=k, T=T, C=C),
        grid_spec=grid_spec,
        out_shape=jax.ShapeDtypeStruct((n, H), jnp.float32),
        compiler_params=pltpu.CompilerParams(
            dimension_semantics=("parallel",)),
        interpret=interpret,
    )(scal, bt3, btr3, bs3, bsc3, xt, xs, u, v, w2, b2)


def _tile_specs(t, shapes):
    specs = [pl.BlockSpec((t, shapes[0][1]), lambda i: (i, 0))]
    for s in shapes[1:]:
        specs.append(pl.BlockSpec(s, lambda i, _s=s: tuple(0 for _ in _s)))
    return specs


def kernel(x_pfc, x_vtx, batch_pfc, batch_vtx,
           vW1, vb1, vW2, vb2, vW3, vb3,
           nW1, nb1, nW2, nb2,
           cW1, cb1, cW2, cb2,
           e1W1, e1b1, e1W2, e1b2,
           e2W1, e2b1, e2W2, e2b2,
           oW1, ob1, oW2, ob2, oW3, ob3):
    k1, k2 = 32, 16
    T = 128
    C = 128

    f32 = jnp.float32
    batch_pfc_i = batch_pfc.astype(jnp.int32)

    x_vtx_enc = pl.pallas_call(
        _vertex_kernel,
        out_shape=jax.ShapeDtypeStruct((x_vtx.shape[0], H), f32),
    )(x_vtx, vW1, vb1[None, :], vW2, vb2[None, :], vW3, vb3[None, :])

    A1 = e1W1[:H] - e1W1[H:]
    B1 = e1W1[H:]
    TP = 512
    enc, u1, v1 = pl.pallas_call(
        _pfc_enc_kernel,
        grid=(N // TP,),
        in_specs=_tile_specs(TP, [(TP, 12), nW1.shape, (1, 32), nW2.shape,
                                  (1, H), cW1.shape, (1, 32), cW2.shape,
                                  (1, H), A1.shape, (1, H), B1.shape]),
        out_specs=[pl.BlockSpec((TP, H), lambda i: (i, 0))] * 3,
        out_shape=[jax.ShapeDtypeStruct((N, H), f32)] * 3,
    )(x_pfc, nW1, nb1[None, :], nW2, nb2[None, :],
      cW1, cb1[None, :], cW2, cb2[None, :],
      A1, e1b1[None, :], B1)

    starts = jnp.searchsorted(batch_pfc_i,
                              jnp.arange(NEV + 1, dtype=jnp.int32)
                              ).astype(jnp.int32)
    ntiles = N // T
    tidx = jnp.arange(ntiles, dtype=jnp.int32)
    first_b = batch_pfc_i[tidx * T]
    last_b = batch_pfc_i[(tidx + 1) * T - 1]
    span_lo = starts[first_b]
    span_hi = starts[last_b + 1]
    lo_row = (span_lo // S) * S
    nblk = (span_hi - lo_row + S - 1) // S
    nblk = jnp.maximum(nblk, 1)
    scal = jnp.stack([lo_row, nblk], axis=1).reshape(-1)

    bt3 = batch_pfc_i.reshape(ntiles, T, 1)
    btr3 = batch_pfc_i.reshape(ntiles, 1, T)
    bs3 = batch_pfc_i.reshape(NBLK, 1, S)
    bsc3 = batch_pfc_i.reshape(NBLK, S, 1)

    feats1 = _edge_conv(scal, bt3, btr3, bs3, bsc3, enc, enc, u1, v1,
                        e1W2, e1b2[None, :], k=k1, T=T, C=C)

    A2f = e2W1[:75] - e2W1[75:]
    B2f = e2W1[75:]
    pad = jnp.zeros((128 - 75, H), f32)
    A2 = jnp.concatenate([A2f, pad], axis=0)
    B2 = jnp.concatenate([B2f, pad], axis=0)
    cat, u2, v2 = pl.pallas_call(
        _prep2_kernel,
        grid=(N // TP,),
        in_specs=[pl.BlockSpec((TP, 12), lambda i: (i, 0)),
                  pl.BlockSpec((TP, H), lambda i: (i, 0)),
                  pl.BlockSpec(A2.shape, lambda i: (0, 0)),
                  pl.BlockSpec((1, H), lambda i: (0, 0)),
                  pl.BlockSpec(B2.shape, lambda i: (0, 0))],
        out_specs=[pl.BlockSpec((TP, 128), lambda i: (i, 0)),
                   pl.BlockSpec((TP, H), lambda i: (i, 0)),
                   pl.BlockSpec((TP, H), lambda i: (i, 0))],
        out_shape=[jax.ShapeDtypeStruct((N, 128), f32),
                   jax.ShapeDtypeStruct((N, H), f32),
                   jax.ShapeDtypeStruct((N, H), f32)],
    )(x_pfc, feats1, A2, e2b1[None, :], B2)

    charged_batch = jnp.where(x_pfc[:, 10] != 0.0, batch_pfc_i,
                              jnp.int32(-1))
    bs3c = charged_batch.reshape(NBLK, 1, S)
    bsc3c = charged_batch.reshape(NBLK, S, 1)
    feats2 = _edge_conv(scal, bt3, btr3, bs3c, bsc3c, cat, cat, u2, v2,
                        e2W2, e2b2[None, :], k=k2, T=T, C=C)

    out_pad = pl.pallas_call(
        _head_kernel,
        grid=(N // TP,),
        in_specs=_tile_specs(TP, [(TP, H), oW1.shape, (1, 16), oW2.shape,
                                  (1, 4), oW3.shape, (1, 1)]),
        out_specs=pl.BlockSpec((TP, 128), lambda i: (i, 0)),
        out_shape=jax.ShapeDtypeStruct((N, 128), f32),
    )(feats2, oW1, ob1[None, :], oW2, ob2[None, :], oW3, ob3[None, :])
    out = out_pad[:, :1]

    return (out, batch_pfc, feats1, x_vtx_enc)
